# baseline (speedup 1.0000x reference)
import functools

import jax
import jax.numpy as jnp
from jax import lax
from jax.experimental import pallas as pl
from jax.experimental.pallas import tpu as pltpu
from jax.experimental.pallas import tpu_sc as plsc

_V = 1000
_NROWS = 1024 * 50
_NC, _NS = 2, 16
_NW = _NC * _NS
_RPW = _NROWS // _NW
_R = 32
_NCHUNK = _RPW // _R


def _scatter_val(buf, idx_v, chunk, val):
    for v in range(_R // 16):
        lr = lax.iota(jnp.int32, 16) + (v * 16)
        col = idx_v[pl.ds(chunk * _R + v * 16, 16)]
        plsc.store_scatter(buf, [lr * _V + col],
                           jnp.full((16,), val, jnp.float32))


@functools.partial(
    pl.kernel,
    out_type=jax.ShapeDtypeStruct((_NROWS * _V,), jnp.float32),
    mesh=plsc.VectorSubcoreMesh(core_axis_name="c", subcore_axis_name="s"),
    scratch_types=[
        pltpu.VMEM((_RPW,), jnp.int32),
        pltpu.VMEM((_R * _V,), jnp.float32),
        pltpu.VMEM((_R * _V,), jnp.float32),
        pltpu.SemaphoreType.DMA,
        pltpu.SemaphoreType.DMA,
    ],
    compiler_params=pltpu.CompilerParams(needs_layout_passes=False),
)
def _onehot_sc(idx_hbm, z_hbm, out_hbm, idx_v, buf0, buf1, sem0, sem1):
    wid = lax.axis_index("s") * _NC + lax.axis_index("c")
    row_base = wid * _RPW
    bufs = (buf0, buf1)
    sems = (sem0, sem1)

    pltpu.sync_copy(idx_hbm.at[pl.ds(row_base, _RPW)], idx_v)
    pltpu.sync_copy(z_hbm, buf0)
    pltpu.sync_copy(z_hbm, buf1)

    def fire(b, chunk):
        _scatter_val(bufs[b], idx_v, chunk, 1.0)
        pltpu.async_copy(
            bufs[b],
            out_hbm.at[pl.ds((row_base + chunk * _R) * _V, _R * _V)],
            sems[b])

    def drain(b):
        pltpu.make_async_copy(
            bufs[b], out_hbm.at[pl.ds(row_base * _V, _R * _V)], sems[b]).wait()

    fire(0, jnp.int32(0))
    fire(1, jnp.int32(1))

    def body(g, carry):
        for b in range(2):
            c = 2 * g + b
            drain(b)
            _scatter_val(bufs[b], idx_v, c - 2, 0.0)
            fire(b, c)
        return carry

    lax.fori_loop(1, _NCHUNK // 2, body, jnp.int32(0))
    drain(0)
    drain(1)


def kernel(input):
    B, L = input.shape
    idx_flat = input.reshape(B * L)
    z = jnp.zeros((_R * _V,), jnp.float32)
    out = _onehot_sc(idx_flat, z)
    return out.reshape(B, L, _V)
